# baseline (speedup 1.0000x reference)
import functools

import jax
import jax.numpy as jnp
from jax import lax
from jax.experimental import pallas as pl
from jax.experimental.pallas import tpu as pltpu
from jax.experimental.pallas import tpu_sc as plsc

MAXREL = 128
TBL = 2 * MAXREL + 1
SEQ = 4096
NBLK = 16
VLEN = 8192
SHIFT0 = SEQ - 1 - MAXREL + 1
TBL_PAD = 272
CT_LO = 31
CT_HI = 33
CSW = 256


def _tec_body(bias_hbm, out_hbm, bias_v, cst0, cst1, stage, cseed0, cseed1,
              v2x, fsem, sem):
    cid = lax.axis_index("c")
    sid = lax.axis_index("s")

    pltpu.sync_copy(bias_hbm, bias_v)

    lanes = lax.iota(jnp.int32, 16)
    big = jnp.full((16,), 3.4e38, jnp.float32)
    v_lo = bias_v[pl.ds(0, 16)]
    v_hi = bias_v[pl.ds(TBL - 1, 16)]
    splat0 = jnp.full((16,), jnp.min(jnp.where(lanes == 0, v_lo, big)))
    splat1 = jnp.full((16,), jnp.min(jnp.where(lanes == 0, v_hi, big)))

    for r in range(8):
        for t in range(CSW // 16):
            cst0[r, pl.ds(t * 16, 16)] = splat0
            cst1[r, pl.ds(t * 16, 16)] = splat1
    stripe = pl.multiple_of(sid * CSW, CSW)
    pltpu.sync_copy(cst0, cseed0.at[:, pl.ds(stripe, CSW)])
    pltpu.sync_copy(cst1, cseed1.at[:, pl.ds(stripe, CSW)])
    plsc.subcore_barrier()

    @pl.when(sid < 8)
    def _build_slab():
        s8 = 16 * sid + 8 * cid
        vrow = pl.multiple_of(8 * sid, 8)
        f0 = pltpu.async_copy(
            cseed0.at[:, pl.ds(0, CT_LO * 128)],
            v2x.at[pl.ds(vrow, 8), pl.ds(0, CT_LO * 128)],
            fsem,
        )
        f1 = pltpu.async_copy(
            cseed1.at[:, pl.ds(0, (63 - CT_HI) * 128)],
            v2x.at[pl.ds(vrow, 8), pl.ds((CT_HI + 1) * 128, (63 - CT_HI) * 128)],
            fsem,
        )
        for ct in range(CT_LO, CT_HI + 1):
            for r in range(8):
                for t in range(8):
                    col = (ct - CT_LO) * 128 + t * 16
                    idx = jnp.clip(
                        ct * 128 + t * 16 + lanes - s8 - r - SHIFT0, 0, TBL - 1
                    )
                    stage[r, pl.ds(col, 16)] = plsc.load_gather(bias_v, [idx])
        pltpu.sync_copy(
            stage,
            v2x.at[pl.ds(vrow, 8), pl.ds(CT_LO * 128, (CT_HI - CT_LO + 1) * 128)],
        )
        f0.wait()
        f1.wait()

    plsc.subcore_barrier()

    trow = pl.multiple_of(8 * (sid % 8), 8)
    kbase = SEQ - 128 * (sid // 8)

    def _blk(b):
        i0 = pl.multiple_of(16 * sid + 8 * cid + 256 * b, 8)
        k0 = pl.multiple_of(kbase - 256 * b, 128)
        return pltpu.make_async_copy(
            v2x.at[pl.ds(trow, 8), pl.ds(k0, SEQ)],
            out_hbm.at[pl.ds(i0, 8), :],
            sem,
        )

    copies = [_blk(b) for b in range(NBLK)]
    for cp in copies:
        cp.start()
    for cp in copies:
        cp.wait()


def _build(bias_pad):
    mesh = plsc.VectorSubcoreMesh(core_axis_name="c", subcore_axis_name="s")
    kern = functools.partial(
        pl.kernel,
        mesh=mesh,
        out_type=jax.ShapeDtypeStruct((SEQ, SEQ), jnp.float32),
        scratch_types=[
            pltpu.VMEM((TBL_PAD,), jnp.float32),
            pltpu.VMEM((8, CSW), jnp.float32),
            pltpu.VMEM((8, CSW), jnp.float32),
            pltpu.VMEM((8, (CT_HI - CT_LO + 1) * 128), jnp.float32),
            pltpu.VMEM_SHARED((8, 16 * CSW), jnp.float32),
            pltpu.VMEM_SHARED((8, 16 * CSW), jnp.float32),
            pltpu.VMEM_SHARED((64, VLEN), jnp.float32),
            pltpu.SemaphoreType.DMA,
            pltpu.SemaphoreType.DMA,
        ],
        compiler_params=pltpu.CompilerParams(needs_layout_passes=False),
    )(_tec_body)
    return kern(bias_pad)


def kernel(x, bias):
    del x
    bias_pad = jnp.zeros((TBL_PAD,), jnp.float32).at[:TBL].set(bias)
    return _build(bias_pad)


# speedup vs baseline: 2.8762x; 2.8762x over previous
import functools

import jax
import jax.numpy as jnp
from jax import lax
from jax.experimental import pallas as pl
from jax.experimental.pallas import tpu as pltpu
from jax.experimental.pallas import tpu_sc as plsc

MAXREL = 128
TBL = 2 * MAXREL + 1
SEQ = 4096
NBLK = 16
VLEN = 8192
SHIFT0 = SEQ - 1 - MAXREL + 1
TBL_PAD = 272
CT_LO = 31
CT_HI = 33
CSW = 256


def _tec_body(bias_hbm, out_hbm, bias_v, cst0, cst1, stage, cseed0, cseed1,
              v2x, fsem, sem):
    cid = lax.axis_index("c")
    sid = lax.axis_index("s")

    pltpu.sync_copy(bias_hbm, bias_v)

    lanes = lax.iota(jnp.int32, 16)
    big = jnp.full((16,), 3.4e38, jnp.float32)
    v_lo = bias_v[pl.ds(0, 16)]
    v_hi = bias_v[pl.ds(TBL - 1, 16)]
    splat0 = jnp.full((16,), jnp.min(jnp.where(lanes == 0, v_lo, big)))
    splat1 = jnp.full((16,), jnp.min(jnp.where(lanes == 0, v_hi, big)))

    for r in range(8):
        for t in range(CSW // 16):
            cst0[r, pl.ds(t * 16, 16)] = splat0
            cst1[r, pl.ds(t * 16, 16)] = splat1
    stripe = pl.multiple_of(sid * CSW, CSW)
    pltpu.sync_copy(cst0, cseed0.at[:, pl.ds(stripe, CSW)])
    pltpu.sync_copy(cst1, cseed1.at[:, pl.ds(stripe, CSW)])
    plsc.subcore_barrier()

    @pl.when(sid < 8)
    def _build_slab():
        s8 = 16 * sid + 8 * cid
        f0 = pltpu.async_copy(
            cseed0.at[:, pl.ds(0, CT_LO * 128)],
            stage.at[:, pl.ds(0, CT_LO * 128)],
            fsem,
        )
        f1 = pltpu.async_copy(
            cseed1.at[:, pl.ds(0, (63 - CT_HI) * 128)],
            stage.at[:, pl.ds((CT_HI + 1) * 128, (63 - CT_HI) * 128)],
            fsem,
        )
        for ct in range(CT_LO, CT_HI + 1):
            for r in range(8):
                for t in range(8):
                    col = ct * 128 + t * 16
                    idx = jnp.clip(col + lanes - s8 - r - SHIFT0, 0, TBL - 1)
                    stage[r, pl.ds(col, 16)] = plsc.load_gather(bias_v, [idx])
        f0.wait()
        f1.wait()
        pltpu.sync_copy(stage, v2x.at[pl.ds(pl.multiple_of(8 * sid, 8), 8), :])

    plsc.subcore_barrier()

    trow = pl.multiple_of(8 * (sid % 8), 8)
    kbase = SEQ - 128 * (sid // 8)

    def _blk(b):
        i0 = pl.multiple_of(16 * sid + 8 * cid + 256 * b, 8)
        k0 = pl.multiple_of(kbase - 256 * b, 128)
        return pltpu.make_async_copy(
            v2x.at[pl.ds(trow, 8), pl.ds(k0, SEQ)],
            out_hbm.at[pl.ds(i0, 8), :],
            sem,
        )

    copies = [_blk(b) for b in range(NBLK)]
    for cp in copies:
        cp.start()
    for cp in copies:
        cp.wait()


def _build(bias_pad):
    mesh = plsc.VectorSubcoreMesh(core_axis_name="c", subcore_axis_name="s")
    kern = functools.partial(
        pl.kernel,
        mesh=mesh,
        out_type=jax.ShapeDtypeStruct((SEQ, SEQ), jnp.float32),
        scratch_types=[
            pltpu.VMEM((TBL_PAD,), jnp.float32),
            pltpu.VMEM((8, CSW), jnp.float32),
            pltpu.VMEM((8, CSW), jnp.float32),
            pltpu.VMEM((8, VLEN), jnp.float32),
            pltpu.VMEM_SHARED((8, 16 * CSW), jnp.float32),
            pltpu.VMEM_SHARED((8, 16 * CSW), jnp.float32),
            pltpu.VMEM_SHARED((64, VLEN), jnp.float32),
            pltpu.SemaphoreType.DMA,
            pltpu.SemaphoreType.DMA,
        ],
        compiler_params=pltpu.CompilerParams(needs_layout_passes=False),
    )(_tec_body)
    return kern(bias_pad)


def kernel(x, bias):
    del x
    bias_pad = jnp.zeros((TBL_PAD,), jnp.float32).at[:TBL].set(bias)
    return _build(bias_pad)


# speedup vs baseline: 2.9759x; 1.0347x over previous
import functools

import jax
import jax.numpy as jnp
from jax import lax
from jax.experimental import pallas as pl
from jax.experimental.pallas import tpu as pltpu
from jax.experimental.pallas import tpu_sc as plsc

MAXREL = 128
TBL = 2 * MAXREL + 1
SEQ = 4096
NBLK = 16
VLEN = 8192
SHIFT0 = SEQ - 1 - MAXREL + 1
TBL_PAD = 272
CT_LO = 31
CT_HI = 33
CSW = 256


def _tec_body(bias_hbm, out_hbm, bias_v, cst0, cst1, stage, cseed0, cseed1,
              v2x, fsem, sem):
    cid = lax.axis_index("c")
    sid = lax.axis_index("s")

    pltpu.sync_copy(bias_hbm, bias_v)

    lanes = lax.iota(jnp.int32, 16)
    big = jnp.full((16,), 3.4e38, jnp.float32)
    v_lo = bias_v[pl.ds(0, 16)]
    v_hi = bias_v[pl.ds(TBL - 1, 16)]
    splat0 = jnp.full((16,), jnp.min(jnp.where(lanes == 0, v_lo, big)))
    splat1 = jnp.full((16,), jnp.min(jnp.where(lanes == 0, v_hi, big)))

    for r in range(8):
        for t in range(CSW // 16):
            cst0[r, pl.ds(t * 16, 16)] = splat0
            cst1[r, pl.ds(t * 16, 16)] = splat1
    stripe = pl.multiple_of(sid * CSW, CSW)
    pltpu.sync_copy(cst0, cseed0.at[:, pl.ds(stripe, CSW)])
    pltpu.sync_copy(cst1, cseed1.at[:, pl.ds(stripe, CSW)])
    plsc.subcore_barrier()

    SPLIT = (CT_HI + 3) * 128
    slab = sid % 8
    s8 = 16 * slab + 8 * cid
    vrow = pl.multiple_of(8 * slab, 8)

    @pl.when(sid < 8)
    def _build_left():
        f0 = pltpu.async_copy(
            cseed0.at[:, pl.ds(0, CT_LO * 128)],
            stage.at[:, pl.ds(0, CT_LO * 128)],
            fsem,
        )
        f1 = pltpu.async_copy(
            cseed1.at[:, pl.ds(0, SPLIT - (CT_HI + 1) * 128)],
            stage.at[:, pl.ds((CT_HI + 1) * 128, SPLIT - (CT_HI + 1) * 128)],
            fsem,
        )
        for ct in range(CT_LO, CT_HI + 1):
            for r in range(8):
                for t in range(8):
                    col = ct * 128 + t * 16
                    idx = jnp.clip(col + lanes - s8 - r - SHIFT0, 0, TBL - 1)
                    stage[r, pl.ds(col, 16)] = plsc.load_gather(bias_v, [idx])
        f0.wait()
        f1.wait()
        pltpu.sync_copy(
            stage.at[:, pl.ds(0, SPLIT)], v2x.at[pl.ds(vrow, 8), pl.ds(0, SPLIT)]
        )

    @pl.when(sid >= 8)
    def _build_right():
        f2 = pltpu.async_copy(
            cseed1.at[:, pl.ds(0, VLEN - SPLIT)],
            stage.at[:, pl.ds(0, VLEN - SPLIT)],
            fsem,
        )
        f2.wait()
        pltpu.sync_copy(
            stage.at[:, pl.ds(0, VLEN - SPLIT)],
            v2x.at[pl.ds(vrow, 8), pl.ds(SPLIT, VLEN - SPLIT)],
        )

    plsc.subcore_barrier()

    trow = pl.multiple_of(8 * (sid % 8), 8)
    kbase = SEQ - 128 * (sid // 8)

    def _blk(b):
        i0 = pl.multiple_of(16 * sid + 8 * cid + 256 * b, 8)
        k0 = pl.multiple_of(kbase - 256 * b, 128)
        return pltpu.make_async_copy(
            v2x.at[pl.ds(trow, 8), pl.ds(k0, SEQ)],
            out_hbm.at[pl.ds(i0, 8), :],
            sem,
        )

    copies = [_blk(b) for b in range(NBLK)]
    for cp in copies:
        cp.start()
    for cp in copies:
        cp.wait()


def _build(bias_pad):
    mesh = plsc.VectorSubcoreMesh(core_axis_name="c", subcore_axis_name="s")
    kern = functools.partial(
        pl.kernel,
        mesh=mesh,
        out_type=jax.ShapeDtypeStruct((SEQ, SEQ), jnp.float32),
        scratch_types=[
            pltpu.VMEM((TBL_PAD,), jnp.float32),
            pltpu.VMEM((8, CSW), jnp.float32),
            pltpu.VMEM((8, CSW), jnp.float32),
            pltpu.VMEM((8, (CT_HI + 3) * 128), jnp.float32),
            pltpu.VMEM_SHARED((8, 16 * CSW), jnp.float32),
            pltpu.VMEM_SHARED((8, 16 * CSW), jnp.float32),
            pltpu.VMEM_SHARED((64, VLEN), jnp.float32),
            pltpu.SemaphoreType.DMA,
            pltpu.SemaphoreType.DMA,
        ],
        compiler_params=pltpu.CompilerParams(needs_layout_passes=False),
    )(_tec_body)
    return kern(bias_pad)


def kernel(x, bias):
    del x
    bias_pad = jnp.zeros((TBL_PAD,), jnp.float32).at[:TBL].set(bias)
    return _build(bias_pad)
